# baseline (speedup 1.0000x reference)
import jax
import jax.numpy as jnp
from jax.experimental import pallas as pl


def _add_pe_kernel(x_ref, pe_ref, o_ref):
    o_ref[...] = x_ref[...] + pe_ref[...][None, :, :]


def kernel(x, pe_table):
    B, L, D = x.shape
    pe = pe_table[:L]
    BB = 32
    return pl.pallas_call(
        _add_pe_kernel,
        grid=(B // BB,),
        in_specs=[
            pl.BlockSpec((BB, L, D), lambda i: (i, 0, 0)),
            pl.BlockSpec((L, D), lambda i: (0, 0)),
        ],
        out_specs=pl.BlockSpec((BB, L, D), lambda i: (i, 0, 0)),
        out_shape=jax.ShapeDtypeStruct((B, L, D), x.dtype),
    )(x, pe)


# speedup vs baseline: 1.0486x; 1.0486x over previous
import jax
import jax.numpy as jnp
from jax.experimental import pallas as pl


def _add_pe_kernel(x_ref, pe_ref, o_ref):
    o_ref[...] = x_ref[...] + pe_ref[...][None, :, :]


def kernel(x, pe_table):
    B, L, D = x.shape
    pe = pe_table[:L]
    BB = 128
    return pl.pallas_call(
        _add_pe_kernel,
        grid=(B // BB,),
        in_specs=[
            pl.BlockSpec((BB, L, D), lambda i: (i, 0, 0)),
            pl.BlockSpec((L, D), lambda i: (0, 0)),
        ],
        out_specs=pl.BlockSpec((BB, L, D), lambda i: (i, 0, 0)),
        out_shape=jax.ShapeDtypeStruct((B, L, D), x.dtype),
    )(x, pe)
